# initial kernel scaffold (unmeasured)
import jax
import jax.numpy as jnp
from jax import lax
from jax.experimental import pallas as pl
from jax.experimental.pallas import tpu as pltpu

N_DEV = 4
M = 8192
D = 2048
CH = M // N_DEV
N_HOPS = 2 * (N_DEV - 1)


def kernel(partial, resid, gamma):
    partial = partial.reshape(M, D)
    gamma = gamma.reshape(1, D)

    def body(p_ref, r_ref, g_ref, o_ref, comm, localf, outv,
             load_sem, store_sem, send_sems, recv_sems):
        i = lax.axis_index("i")
        right = lax.rem(i + 1, N_DEV)
        left = lax.rem(i + N_DEV - 1, N_DEV)

        barrier = pltpu.get_barrier_semaphore()
        for nbr in (left, right):
            pl.semaphore_signal(
                barrier, inc=1, device_id=(nbr,),
                device_id_type=pl.DeviceIdType.MESH,
            )
        pl.semaphore_wait(barrier, 2)

        def load(src_ref, c, dst):
            cp = pltpu.make_async_copy(
                src_ref.at[pl.ds(c * CH, CH), :], dst, load_sem)
            cp.start()
            cp.wait()

        def hop(g):
            rdma = pltpu.make_async_remote_copy(
                src_ref=comm.at[g % 2],
                dst_ref=comm.at[(g + 1) % 2],
                send_sem=send_sems.at[g],
                recv_sem=recv_sems.at[g],
                device_id=(right,),
                device_id_type=pl.DeviceIdType.MESH,
            )
            rdma.start()
            rdma.wait()

        def chunk_of(s):
            return lax.rem(i - s + 2 * N_DEV, N_DEV)

        load(p_ref, chunk_of(0), localf)
        comm[0] = localf[...].astype(jnp.bfloat16)
        hop(0)
        for s in (1, 2):
            load(p_ref, chunk_of(s), localf)
            slot = s % 2
            comm[slot] = comm[slot] + localf[...].astype(jnp.bfloat16)
            hop(s)

        o = lax.rem(i + 1, N_DEV)
        load(p_ref, o, localf)
        comm[1] = comm[1] + localf[...].astype(jnp.bfloat16)

        load(r_ref, o, localf)
        outv[...] = comm[1][...].astype(jnp.float32) + localf[...]
        ms = jnp.mean(outv[...] * outv[...], axis=1, keepdims=True)
        outv[...] = outv[...] * lax.rsqrt(ms + 1e-6) * g_ref[...]
        comm[1] = outv[...].astype(jnp.bfloat16)

        def store(c):
            cp = pltpu.make_async_copy(
                outv, o_ref.at[pl.ds(c * CH, CH), :], store_sem)
            cp.start()
            cp.wait()

        store(o)

        for h in range(N_DEV - 1):
            g = 3 + h
            hop(g)
            origin = lax.rem(i - h + 2 * N_DEV, N_DEV)
            outv[...] = comm[(g + 1) % 2][...].astype(jnp.float32)
            store(origin)

    return pl.pallas_call(
        body,
        out_shape=jax.ShapeDtypeStruct((M, D), jnp.float32),
        in_specs=[
            pl.BlockSpec(memory_space=pltpu.ANY),
            pl.BlockSpec(memory_space=pltpu.ANY),
            pl.BlockSpec(memory_space=pltpu.VMEM),
        ],
        out_specs=pl.BlockSpec(memory_space=pltpu.ANY),
        scratch_shapes=[
            pltpu.VMEM((2, CH, D), jnp.bfloat16),
            pltpu.VMEM((CH, D), jnp.float32),
            pltpu.VMEM((CH, D), jnp.float32),
            pltpu.SemaphoreType.DMA,
            pltpu.SemaphoreType.DMA,
            pltpu.SemaphoreType.DMA((N_HOPS,)),
            pltpu.SemaphoreType.DMA((N_HOPS,)),
        ],
        compiler_params=pltpu.CompilerParams(collective_id=0),
    )(partial, resid, gamma)


# baseline (device time: 661742 ns/iter reference)
import jax
import jax.numpy as jnp
from jax import lax
from jax.experimental import pallas as pl
from jax.experimental.pallas import tpu as pltpu

N_DEV = 4
M = 8192
D = 2048
CH = M // N_DEV
N_HOPS = 2 * (N_DEV - 1)


def kernel(partial, resid, gamma):
    partial = partial.reshape(M, D)
    gamma = gamma.reshape(1, D)

    def body(p_ref, r_ref, g_ref, o_ref, comm, localf,
             load_sem, store_sem, send_sems, recv_sems):
        i = lax.axis_index("i")
        right = lax.rem(i + 1, N_DEV)
        left = lax.rem(i + N_DEV - 1, N_DEV)

        barrier = pltpu.get_barrier_semaphore()
        for nbr in (left, right):
            pl.semaphore_signal(
                barrier, inc=1, device_id=(nbr,),
                device_id_type=pl.DeviceIdType.MESH,
            )
        pl.semaphore_wait(barrier, 2)

        def load(src_ref, c, dst):
            cp = pltpu.make_async_copy(
                src_ref.at[pl.ds(c * CH, CH), :], dst, load_sem)
            cp.start()
            cp.wait()

        def hop(g):
            rdma = pltpu.make_async_remote_copy(
                src_ref=comm.at[g % 2],
                dst_ref=comm.at[(g + 1) % 2],
                send_sem=send_sems.at[g],
                recv_sem=recv_sems.at[g],
                device_id=(right,),
                device_id_type=pl.DeviceIdType.MESH,
            )
            rdma.start()
            rdma.wait()

        def chunk_of(s):
            return lax.rem(i - s + 2 * N_DEV, N_DEV)

        load(p_ref, chunk_of(0), localf)
        comm[0] = localf[...].astype(jnp.bfloat16)
        hop(0)
        for s in (1, 2):
            load(p_ref, chunk_of(s), localf)
            slot = s % 2
            comm[slot] = comm[slot] + localf[...].astype(jnp.bfloat16)
            hop(s)

        o = lax.rem(i + 1, N_DEV)
        load(p_ref, o, localf)
        comm[1] = comm[1] + localf[...].astype(jnp.bfloat16)

        load(r_ref, o, localf)
        localf[...] = comm[1][...].astype(jnp.float32) + localf[...]
        ms = jnp.mean(localf[...] * localf[...], axis=1, keepdims=True)
        localf[...] = localf[...] * lax.rsqrt(ms + 1e-6) * g_ref[...]
        comm[1] = localf[...].astype(jnp.bfloat16)

        def store(c):
            cp = pltpu.make_async_copy(
                localf, o_ref.at[pl.ds(c * CH, CH), :], store_sem)
            cp.start()
            cp.wait()

        store(o)

        for h in range(N_DEV - 1):
            g = 3 + h
            hop(g)
            origin = lax.rem(i - h + 2 * N_DEV, N_DEV)
            localf[...] = comm[(g + 1) % 2][...].astype(jnp.float32)
            store(origin)

    return pl.pallas_call(
        body,
        out_shape=jax.ShapeDtypeStruct((M, D), jnp.float32),
        in_specs=[
            pl.BlockSpec(memory_space=pl.ANY),
            pl.BlockSpec(memory_space=pl.ANY),
            pl.BlockSpec(memory_space=pltpu.VMEM),
        ],
        out_specs=pl.BlockSpec(memory_space=pl.ANY),
        scratch_shapes=[
            pltpu.VMEM((2, CH, D), jnp.bfloat16),
            pltpu.VMEM((CH, D), jnp.float32),
            pltpu.SemaphoreType.DMA,
            pltpu.SemaphoreType.DMA,
            pltpu.SemaphoreType.DMA((N_HOPS,)),
            pltpu.SemaphoreType.DMA((N_HOPS,)),
        ],
        compiler_params=pltpu.CompilerParams(
            collective_id=0, vmem_limit_bytes=64 * 1024 * 1024),
    )(partial, resid, gamma)


# device time: 355958 ns/iter; 1.8590x vs baseline; 1.8590x over previous
import jax
import jax.numpy as jnp
from jax import lax
from jax.experimental import pallas as pl
from jax.experimental.pallas import tpu as pltpu

N_DEV = 4
M = 8192
D = 2048
CH = M // N_DEV
H = CH // 2
N_HOPS = 2 * (N_DEV - 1)
TR = 512


def kernel(partial, resid, gamma):
    partial = partial.reshape(M, D)
    gamma = gamma.reshape(1, D)

    def body(p_ref, r_ref, g_ref, o_ref, comm, stage,
             load_sems, store_sems, send_sems, recv_sems):
        i = lax.axis_index("i")
        right = lax.rem(i + 1, N_DEV)
        left = lax.rem(i + N_DEV - 1, N_DEV)

        barrier = pltpu.get_barrier_semaphore()
        for nbr in (left, right):
            pl.semaphore_signal(
                barrier, inc=1, device_id=(nbr,),
                device_id_type=pl.DeviceIdType.MESH,
            )
        pl.semaphore_wait(barrier, 2)

        def mod(v):
            return lax.rem(v + 2 * N_DEV, N_DEV)

        def load(src_ref, c, d):
            cp = pltpu.make_async_copy(
                src_ref.at[pl.ds(c * CH + d * H, H), :], stage.at[d],
                load_sems.at[d])
            cp.start()
            return cp

        def store(d, c):
            cp = pltpu.make_async_copy(
                stage.at[d], o_ref.at[pl.ds(c * CH + d * H, H), :],
                store_sems.at[d])
            cp.start()
            return cp

        def hop(g):
            out = []
            for d, tgt in ((0, right), (1, left)):
                rdma = pltpu.make_async_remote_copy(
                    src_ref=comm.at[d, g % 2],
                    dst_ref=comm.at[d, (g + 1) % 2],
                    send_sem=send_sems.at[d, g],
                    recv_sem=recv_sems.at[d, g],
                    device_id=(tgt,),
                    device_id_type=pl.DeviceIdType.MESH,
                )
                rdma.start()
                out.append(rdma)
            return out

        def cvt_to_comm(d, slot):
            for r in range(0, H, TR):
                comm[d, slot, r:r + TR] = (
                    stage[d, r:r + TR].astype(jnp.bfloat16))

        def add_to_comm(d, slot):
            for r in range(0, H, TR):
                comm[d, slot, r:r + TR] = (
                    comm[d, slot, r:r + TR]
                    + stage[d, r:r + TR].astype(jnp.bfloat16))

        o_cw = mod(i + 1)
        o_ccw = mod(i - 1)

        for ld in [load(p_ref, i, 0), load(p_ref, i, 1)]:
            ld.wait()
        cvt_to_comm(0, 0)
        cvt_to_comm(1, 0)

        for s in range(N_DEV - 1):
            rdmas = hop(s)
            if s < N_DEV - 2:
                loads = [load(p_ref, mod(i - s - 1), 0),
                         load(p_ref, mod(i + s + 1), 1)]
            else:
                loads = [load(p_ref, o_cw, 0), load(p_ref, o_ccw, 1)]
            for ld in loads:
                ld.wait()
            for r in rdmas:
                r.wait()
            add_to_comm(0, (s + 1) % 2)
            add_to_comm(1, (s + 1) % 2)

        for ld in [load(r_ref, o_cw, 0), load(r_ref, o_ccw, 1)]:
            ld.wait()
        for d in (0, 1):
            for r in range(0, H, TR):
                y = (comm[d, 1, r:r + TR].astype(jnp.float32)
                     + stage[d, r:r + TR])
                ms = jnp.mean(y * y, axis=1, keepdims=True)
                z = y * lax.rsqrt(ms + 1e-6) * g_ref[...]
                stage[d, r:r + TR] = z
                comm[d, 1, r:r + TR] = z.astype(jnp.bfloat16)

        rdmas = hop(3)
        st = {0: store(0, o_cw), 1: store(1, o_ccw)}
        for h in range(N_DEV - 1):
            g = 3 + h
            for r in rdmas:
                r.wait()
            if h < N_DEV - 2:
                rdmas = hop(g + 1)
            for d in (0, 1):
                st.pop(d).wait()
                for r in range(0, H, TR):
                    stage[d, r:r + TR] = (
                        comm[d, (g + 1) % 2, r:r + TR].astype(jnp.float32))
            st = {0: store(0, mod(i - h)), 1: store(1, mod(i + h))}
        for cp in st.values():
            cp.wait()

    return pl.pallas_call(
        body,
        out_shape=jax.ShapeDtypeStruct((M, D), jnp.float32),
        in_specs=[
            pl.BlockSpec(memory_space=pl.ANY),
            pl.BlockSpec(memory_space=pl.ANY),
            pl.BlockSpec(memory_space=pltpu.VMEM),
        ],
        out_specs=pl.BlockSpec(memory_space=pl.ANY),
        scratch_shapes=[
            pltpu.VMEM((2, 2, H, D), jnp.bfloat16),
            pltpu.VMEM((2, H, D), jnp.float32),
            pltpu.SemaphoreType.DMA((2,)),
            pltpu.SemaphoreType.DMA((2,)),
            pltpu.SemaphoreType.DMA((2, N_HOPS)),
            pltpu.SemaphoreType.DMA((2, N_HOPS)),
        ],
        compiler_params=pltpu.CompilerParams(
            collective_id=0, vmem_limit_bytes=64 * 1024 * 1024),
    )(partial, resid, gamma)


# device time: 332524 ns/iter; 1.9901x vs baseline; 1.0705x over previous
import jax
import jax.numpy as jnp
from jax import lax
from jax.experimental import pallas as pl
from jax.experimental.pallas import tpu as pltpu

N_DEV = 4
M = 8192
D = 2048
CH = M // N_DEV
H = CH // 2
NSUB = 2
SUB = H // NSUB
N_HOPS = 2 * (N_DEV - 1)


def kernel(partial, resid, gamma):
    partial = partial.reshape(M, D)
    gamma = gamma.reshape(1, D)

    def body(p_ref, r_ref, g_ref, o_ref, comm, stage,
             load_sems, store_sems, send_sems, recv_sems):
        i = lax.axis_index("i")
        right = lax.rem(i + 1, N_DEV)
        left = lax.rem(i + N_DEV - 1, N_DEV)

        barrier = pltpu.get_barrier_semaphore()
        for nbr in (left, right):
            pl.semaphore_signal(
                barrier, inc=1, device_id=(nbr,),
                device_id_type=pl.DeviceIdType.MESH,
            )
        pl.semaphore_wait(barrier, 2)

        def mod(v):
            return lax.rem(v + 2 * N_DEV, N_DEV)

        def load_half(src_ref, c, d):
            cp = pltpu.make_async_copy(
                src_ref.at[pl.ds(c * CH + d * H, H), :], stage.at[d],
                load_sems.at[d, 0])
            cp.start()
            return cp

        def load_sub(src_ref, c, d, k):
            cp = pltpu.make_async_copy(
                src_ref.at[pl.ds(c * CH + d * H + k * SUB, SUB), :],
                stage.at[d, pl.ds(k * SUB, SUB)],
                load_sems.at[d, k])
            cp.start()
            return cp

        def store_sub(d, c, k):
            cp = pltpu.make_async_copy(
                stage.at[d, pl.ds(k * SUB, SUB)],
                o_ref.at[pl.ds(c * CH + d * H + k * SUB, SUB), :],
                store_sems.at[d, k])
            cp.start()
            return cp

        def rdma_sub(d, g, k):
            rdma = pltpu.make_async_remote_copy(
                src_ref=comm.at[d, g % 2, pl.ds(k * SUB, SUB)],
                dst_ref=comm.at[d, (g + 1) % 2, pl.ds(k * SUB, SUB)],
                send_sem=send_sems.at[d, g, k],
                recv_sem=recv_sems.at[d, g, k],
                device_id=(right if d == 0 else left,),
                device_id_type=pl.DeviceIdType.MESH,
            )
            rdma.start()
            return rdma

        o_cw = mod(i + 1)
        o_ccw = mod(i - 1)

        def rows(k):
            return slice(k * SUB, (k + 1) * SUB)

        for ld in [load_half(p_ref, i, 0), load_half(p_ref, i, 1)]:
            ld.wait()
        inflight = {}
        for k in range(NSUB):
            for d in (0, 1):
                comm[d, 0, rows(k)] = stage[d, rows(k)].astype(jnp.bfloat16)
                inflight[(d, k)] = rdma_sub(d, 0, k)
        loads = [load_half(p_ref, mod(i - 1), 0), load_half(p_ref, mod(i + 1), 1)]

        resid_loads = {}
        for s in range(N_DEV - 1):
            slot = (s + 1) % 2
            for ld in loads:
                ld.wait()
            loads = []
            nxt = {}
            for k in range(NSUB):
                for d in (0, 1):
                    inflight.pop((d, k)).wait()
                    comm[d, slot, rows(k)] = (
                        comm[d, slot, rows(k)]
                        + stage[d, rows(k)].astype(jnp.bfloat16))
                    if s < N_DEV - 2:
                        nxt[(d, k)] = rdma_sub(d, s + 1, k)
                    else:
                        resid_loads[(d, k)] = load_sub(
                            r_ref, o_cw if d == 0 else o_ccw, d, k)
            inflight = nxt
            if s == 0:
                loads = [load_half(p_ref, mod(i - 2), 0),
                         load_half(p_ref, mod(i + 2), 1)]
            elif s == 1:
                loads = [load_half(p_ref, o_cw, 0), load_half(p_ref, o_ccw, 1)]

        stores = {}
        for k in range(NSUB):
            for d in (0, 1):
                resid_loads.pop((d, k)).wait()
                y = comm[d, 1, rows(k)].astype(jnp.float32) + stage[d, rows(k)]
                ms = jnp.mean(y * y, axis=1, keepdims=True)
                z = y * lax.rsqrt(ms + 1e-6) * g_ref[...]
                stage[d, rows(k)] = z
                comm[d, 1, rows(k)] = z.astype(jnp.bfloat16)
                inflight[(d, k)] = rdma_sub(d, 3, k)
                stores[(d, k)] = store_sub(d, o_cw if d == 0 else o_ccw, k)

        for h in range(N_DEV - 1):
            g = 3 + h
            nxt = {}
            for k in range(NSUB):
                for d in (0, 1):
                    inflight.pop((d, k)).wait()
                    if h < N_DEV - 2:
                        nxt[(d, k)] = rdma_sub(d, g + 1, k)
                    stores.pop((d, k)).wait()
                    stage[d, rows(k)] = (
                        comm[d, (g + 1) % 2, rows(k)].astype(jnp.float32))
                    stores[(d, k)] = store_sub(
                        d, mod(i - h) if d == 0 else mod(i + h), k)
            inflight = nxt
        for cp in stores.values():
            cp.wait()

    return pl.pallas_call(
        body,
        out_shape=jax.ShapeDtypeStruct((M, D), jnp.float32),
        in_specs=[
            pl.BlockSpec(memory_space=pl.ANY),
            pl.BlockSpec(memory_space=pl.ANY),
            pl.BlockSpec(memory_space=pltpu.VMEM),
        ],
        out_specs=pl.BlockSpec(memory_space=pl.ANY),
        scratch_shapes=[
            pltpu.VMEM((2, 2, H, D), jnp.bfloat16),
            pltpu.VMEM((2, H, D), jnp.float32),
            pltpu.SemaphoreType.DMA((2, NSUB)),
            pltpu.SemaphoreType.DMA((2, NSUB)),
            pltpu.SemaphoreType.DMA((2, N_HOPS, NSUB)),
            pltpu.SemaphoreType.DMA((2, N_HOPS, NSUB)),
        ],
        compiler_params=pltpu.CompilerParams(
            collective_id=0, vmem_limit_bytes=64 * 1024 * 1024),
    )(partial, resid, gamma)
